# baseline (speedup 1.0000x reference)
import functools

import jax
import jax.numpy as jnp
from jax import lax
from jax.experimental import pallas as pl
from jax.experimental.pallas import tpu as pltpu
from jax.experimental.pallas import tpu_sc as plsc

N_NODES = 10000
F_IN = 128
C1 = 16
C2 = 7
NC = 2
NS = 16
NW = NC * NS
B = 128
NBUF = 4
ROWS_PER_TILE = N_NODES // NS


NF = N_NODES // 8


def _block_diag(w, blocks, rows, cols):
    t = jnp.tile(w, (blocks, blocks))
    i = lax.broadcasted_iota(jnp.int32, (blocks * rows, blocks * cols), 0)
    j = lax.broadcasted_iota(jnp.int32, (blocks * rows, blocks * cols), 1)
    return jnp.where((i // rows) == (j // cols), t, 0.0)


def _xw1_body(x_ref, w_ref, p0_ref, p1_ref):
    x3 = x_ref[...].reshape(NF, 8, F_IN)
    for k in range(8):
        xk = x3[:, k, :]
        p0_ref[:, k * C1:(k + 1) * C1] = jnp.dot(
            xk, w_ref[0], preferred_element_type=jnp.float32)
        p1_ref[:, k * C1:(k + 1) * C1] = jnp.dot(
            xk, w_ref[1], preferred_element_type=jnp.float32)


def _finish1_body(p0_ref, s0_ref, s1_ref, w2_ref, h_ref, q0_ref):
    h = jnp.maximum(p0_ref[...] + s0_ref[...] + s1_ref[...], 0.0)
    h_ref[...] = h
    w2bd = _block_diag(w2_ref[0], 8, C1, C2)
    q0_ref[...] = jnp.dot(h.reshape(NF, 128), w2bd,
                          preferred_element_type=jnp.float32)


def _finish2_body(q0_ref, s0_ref, s1_ref, w21_ref, out_ref):
    sf = (s0_ref[...] + s1_ref[...]).reshape(NF, 128)
    w21bd = _block_diag(w21_ref[...], 8, C1, C2)
    z = q0_ref[...] + jnp.dot(sf, w21bd, preferred_element_type=jnp.float32)
    m = jnp.max(z, axis=-1, keepdims=True)
    e = jnp.exp(z - m)
    ones7 = jnp.ones((C2, C2), jnp.float32)
    denom = jnp.dot(e, _block_diag(ones7, 8, C2, C2),
                    preferred_element_type=jnp.float32)
    out_ref[...] = e / denom


_tc_xw1 = pl.pallas_call(
    _xw1_body,
    out_shape=[jax.ShapeDtypeStruct((NF, 128), jnp.float32),
               jax.ShapeDtypeStruct((NF, 128), jnp.float32)],
)

_tc_finish1 = pl.pallas_call(
    _finish1_body,
    out_shape=[jax.ShapeDtypeStruct((N_NODES * C1,), jnp.float32),
               jax.ShapeDtypeStruct((NF, 8 * C2), jnp.float32)],
)

_tc_finish2 = pl.pallas_call(
    _finish2_body,
    out_shape=jax.ShapeDtypeStruct((NF, 8 * C2), jnp.float32),
)


def _wsplat(w16, e2):
    return lax.gather(
        w16, jnp.full((16, 1), e2, jnp.int32),
        dimension_numbers=lax.GatherDimensionNumbers(
            offset_dims=(), collapsed_slice_dims=(0,), start_index_map=(0,)),
        slice_sizes=(1,),
        mode=lax.GatherScatterMode.PROMISE_IN_BOUNDS)


def _spmm_body(n_chunks, table_ref, e3_ref, w_ref,
               out0_ref, out1_ref,
               table_sh, acc, eidx, wv, rows, srows, zbuf, gsem, ssem):
    cid = lax.axis_index("c")
    sid = lax.axis_index("s")
    wid = cid * NS + sid
    ew = n_chunks * B
    n_groups = n_chunks // NBUF

    cp_e = pltpu.async_copy(e3_ref.at[pl.ds(wid * n_chunks, n_chunks)], eidx,
                            gsem.at[0])
    cp_w = pltpu.async_copy(w_ref.at[pl.ds(wid * ew, ew)], wv, gsem.at[1])
    row0 = sid * ROWS_PER_TILE
    cp_t = pltpu.async_copy(table_ref.at[pl.ds(row0, ROWS_PER_TILE)],
                            table_sh.at[pl.ds(row0, ROWS_PER_TILE)], gsem.at[2])

    def _zinit(i, carry):
        zbuf[i, :] = jnp.zeros((C1,), jnp.float32)
        return carry
    lax.fori_loop(0, ROWS_PER_TILE, _zinit, 0)
    pltpu.sync_copy(zbuf, acc.at[pl.ds(row0, ROWS_PER_TILE)])
    cp_e.wait()
    cp_w.wait()
    cp_t.wait()
    plsc.subcore_barrier()

    for b in range(NBUF):
        pltpu.async_copy(table_sh.at[eidx.at[b, 0]], rows.at[b], gsem.at[b])

    def _group(g, carry):
        for b in range(NBUF):
            c = g * NBUF + b

            @pl.when(g > 0)
            def _():
                pltpu.make_async_copy(
                    srows.at[b], acc.at[eidx.at[c, 1]], ssem.at[b]).wait()

            pltpu.make_async_copy(
                table_sh.at[eidx.at[c, 0]], rows.at[b], gsem.at[b]).wait()

            def _scale(g2, carry2):
                w16 = wv[pl.ds(c * B + g2 * 16, 16)]
                base = g2 * 16
                for e2 in range(16):
                    srows[b, base + e2, :] = rows[b, base + e2, :] * _wsplat(w16, e2)
                return carry2
            lax.fori_loop(0, B // 16, _scale, 0)

            pltpu.async_copy(srows.at[b], acc.at[eidx.at[c, 1]], ssem.at[b],
                             add=True)

            @pl.when(g + 1 < n_groups)
            def _():
                pltpu.async_copy(table_sh.at[eidx.at[c + NBUF, 0]],
                                 rows.at[b], gsem.at[b])
        return carry
    lax.fori_loop(0, n_groups, _group, 0)

    for b in range(NBUF):
        c = (n_groups - 1) * NBUF + b
        pltpu.make_async_copy(srows.at[b], acc.at[eidx.at[c, 1]], ssem.at[b]).wait()

    plsc.subcore_barrier()

    @pl.when(cid == 0)
    def _():
        pltpu.sync_copy(acc.at[pl.ds(row0, ROWS_PER_TILE)],
                        out0_ref.at[pl.ds(row0, ROWS_PER_TILE)])

    @pl.when(cid == 1)
    def _():
        pltpu.sync_copy(acc.at[pl.ds(row0, ROWS_PER_TILE)],
                        out1_ref.at[pl.ds(row0, ROWS_PER_TILE)])


def _spmm(table, e3, w_flat):
    n_chunks = e3.shape[0] // NW
    mesh = plsc.VectorSubcoreMesh(core_axis_name="c", subcore_axis_name="s")
    f = pl.kernel(
        functools.partial(_spmm_body, n_chunks),
        out_type=[jax.ShapeDtypeStruct((N_NODES, C1), jnp.float32),
                  jax.ShapeDtypeStruct((N_NODES, C1), jnp.float32)],
        mesh=mesh,
        scratch_types=[
            pltpu.VMEM_SHARED((N_NODES, C1), jnp.float32),
            pltpu.VMEM_SHARED((N_NODES, C1), jnp.float32),
            pltpu.VMEM((n_chunks, 2, B), jnp.int32),
            pltpu.VMEM((n_chunks * B,), jnp.float32),
            pltpu.VMEM((NBUF, B, C1), jnp.float32),
            pltpu.VMEM((NBUF, B, C1), jnp.float32),
            pltpu.VMEM((ROWS_PER_TILE, C1), jnp.float32),
            pltpu.SemaphoreType.DMA((NBUF,)),
            pltpu.SemaphoreType.DMA((NBUF,)),
        ],
        compiler_params=pltpu.CompilerParams(use_tc_tiling_on_sc=False),
    )
    return f(table, e3, w_flat)


def kernel(x, edge_index, edge_weight, W1, W2):
    e_total = edge_weight.shape[0]
    per = NW * B
    n_chunks = -(-e_total // per)
    n_chunks = -(-n_chunks // NBUF) * NBUF
    pad = n_chunks * per - e_total

    ei = edge_index.astype(jnp.int32)
    w = edge_weight.astype(jnp.float32)
    if pad:
        pad_blk = jnp.stack([jnp.zeros((pad,), jnp.int32),
                             jnp.arange(pad, dtype=jnp.int32) % N_NODES])
        ei = jnp.concatenate([ei, pad_blk], axis=1)
        w = jnp.pad(w, (0, pad))
    e3 = ei.reshape(2, NW * n_chunks, B).transpose(1, 0, 2)

    p0l, p1l = _tc_xw1(x, W1)
    s1a, s1b = _spmm(p1l.reshape(N_NODES, C1), e3, w)
    hl, q0f = _tc_finish1(p0l.reshape(-1), s1a.reshape(-1), s1b.reshape(-1), W2)
    s2a, s2b = _spmm(hl.reshape(N_NODES, C1), e3, w)
    outf = _tc_finish2(q0f, s2a.reshape(-1), s2b.reshape(-1), W2[1])
    return outf.reshape(N_NODES, C2)


# speedup vs baseline: 1.2226x; 1.2226x over previous
import functools

import jax
import jax.numpy as jnp
from jax import lax
from jax.experimental import pallas as pl
from jax.experimental.pallas import tpu as pltpu
from jax.experimental.pallas import tpu_sc as plsc

N_NODES = 10000
F_IN = 128
C1 = 16
C2 = 7
NC = 2
NS = 16
NW = NC * NS
B = 128
NBUF = 4
ROWS_PER_TILE = N_NODES // NS


NF = N_NODES // 8


def _block_diag(w, blocks, rows, cols):
    t = jnp.tile(w, (blocks, blocks))
    i = lax.broadcasted_iota(jnp.int32, (blocks * rows, blocks * cols), 0)
    j = lax.broadcasted_iota(jnp.int32, (blocks * rows, blocks * cols), 1)
    return jnp.where((i // rows) == (j // cols), t, 0.0)


def _xw1_body(x_ref, w_ref, p0_ref, p1_ref):
    x3 = x_ref[...].reshape(NF, 8, F_IN)
    for k in range(8):
        xk = x3[:, k, :]
        p0_ref[:, k * C1:(k + 1) * C1] = jnp.dot(
            xk, w_ref[0], preferred_element_type=jnp.float32)
        p1_ref[:, k * C1:(k + 1) * C1] = jnp.dot(
            xk, w_ref[1], preferred_element_type=jnp.float32)


def _finish1_body(p0_ref, s0_ref, s1_ref, w2_ref, h_ref, q0_ref):
    h = jnp.maximum(p0_ref[...] + s0_ref[...] + s1_ref[...], 0.0)
    h_ref[...] = h
    w2bd = _block_diag(w2_ref[0], 8, C1, C2)
    q0_ref[...] = jnp.dot(h.reshape(NF, 128), w2bd,
                          preferred_element_type=jnp.float32)


def _finish2_body(q0_ref, s0_ref, s1_ref, w21_ref, out_ref):
    sf = (s0_ref[...] + s1_ref[...]).reshape(NF, 128)
    w21bd = _block_diag(w21_ref[...], 8, C1, C2)
    z = q0_ref[...] + jnp.dot(sf, w21bd, preferred_element_type=jnp.float32)
    m = jnp.max(z, axis=-1, keepdims=True)
    e = jnp.exp(z - m)
    ones7 = jnp.ones((C2, C2), jnp.float32)
    denom = jnp.dot(e, _block_diag(ones7, 8, C2, C2),
                    preferred_element_type=jnp.float32)
    out_ref[...] = e / denom


_tc_xw1 = pl.pallas_call(
    _xw1_body,
    out_shape=[jax.ShapeDtypeStruct((NF, 128), jnp.float32),
               jax.ShapeDtypeStruct((NF, 128), jnp.float32)],
)

_tc_finish1 = pl.pallas_call(
    _finish1_body,
    out_shape=[jax.ShapeDtypeStruct((N_NODES * C1,), jnp.float32),
               jax.ShapeDtypeStruct((NF, 8 * C2), jnp.float32)],
)

_tc_finish2 = pl.pallas_call(
    _finish2_body,
    out_shape=jax.ShapeDtypeStruct((NF, 8 * C2), jnp.float32),
)


def _wsplat(w16, e2):
    return lax.gather(
        w16, jnp.full((16, 1), e2, jnp.int32),
        dimension_numbers=lax.GatherDimensionNumbers(
            offset_dims=(), collapsed_slice_dims=(0,), start_index_map=(0,)),
        slice_sizes=(1,),
        mode=lax.GatherScatterMode.PROMISE_IN_BOUNDS)


def _spmm_body(nc_max, tail, table_ref, e3_ref, w_ref,
               out0_ref, out1_ref,
               table_sh, acc, eidx, wv, rows, srows, zbuf, gsem, ssem):
    cid = lax.axis_index("c")
    sid = lax.axis_index("s")
    wid = cid * NS + sid
    last = NW - 1
    nc_local = jnp.where(wid == last, tail, nc_max)
    n_groups = nc_local // NBUF

    @pl.when(wid != last)
    def _():
        pltpu.async_copy(e3_ref.at[pl.ds(wid * nc_max, nc_max)], eidx,
                         gsem.at[0])
        pltpu.async_copy(w_ref.at[pl.ds(wid * nc_max * B, nc_max * B)], wv,
                         gsem.at[1])

    @pl.when(wid == last)
    def _():
        pltpu.async_copy(e3_ref.at[pl.ds(last * nc_max, tail)],
                         eidx.at[pl.ds(0, tail)], gsem.at[0])
        pltpu.async_copy(w_ref.at[pl.ds(last * nc_max * B, tail * B)],
                         wv.at[pl.ds(0, tail * B)], gsem.at[1])

    row0 = sid * ROWS_PER_TILE
    cp_t = pltpu.async_copy(table_ref.at[pl.ds(row0, ROWS_PER_TILE)],
                            table_sh.at[pl.ds(row0, ROWS_PER_TILE)], gsem.at[2])

    def _zinit(i, carry):
        zbuf[i, :] = jnp.zeros((C1,), jnp.float32)
        return carry
    lax.fori_loop(0, ROWS_PER_TILE, _zinit, 0)
    pltpu.sync_copy(zbuf, acc.at[pl.ds(row0, ROWS_PER_TILE)])

    @pl.when(wid != last)
    def _():
        pltpu.make_async_copy(e3_ref.at[pl.ds(wid * nc_max, nc_max)], eidx,
                              gsem.at[0]).wait()
        pltpu.make_async_copy(w_ref.at[pl.ds(wid * nc_max * B, nc_max * B)],
                              wv, gsem.at[1]).wait()

    @pl.when(wid == last)
    def _():
        pltpu.make_async_copy(e3_ref.at[pl.ds(last * nc_max, tail)],
                              eidx.at[pl.ds(0, tail)], gsem.at[0]).wait()
        pltpu.make_async_copy(w_ref.at[pl.ds(last * nc_max * B, tail * B)],
                              wv.at[pl.ds(0, tail * B)], gsem.at[1]).wait()

    cp_t.wait()
    plsc.subcore_barrier()

    for b in range(NBUF):
        pltpu.async_copy(table_sh.at[eidx.at[b, 0]], rows.at[b], gsem.at[b])

    def _group(g, carry):
        for b in range(NBUF):
            c = g * NBUF + b

            @pl.when(g > 0)
            def _():
                pltpu.make_async_copy(
                    srows.at[b], acc.at[eidx.at[c, 1]], ssem.at[b]).wait()

            pltpu.make_async_copy(
                table_sh.at[eidx.at[c, 0]], rows.at[b], gsem.at[b]).wait()

            def _scale(g2, carry2):
                w16 = wv[pl.ds(c * B + g2 * 16, 16)]
                base = g2 * 16
                for e2 in range(16):
                    srows[b, base + e2, :] = rows[b, base + e2, :] * _wsplat(w16, e2)
                return carry2
            lax.fori_loop(0, B // 16, _scale, 0)

            pltpu.async_copy(srows.at[b], acc.at[eidx.at[c, 1]], ssem.at[b],
                             add=True)

            @pl.when(g + 1 < n_groups)
            def _():
                pltpu.async_copy(table_sh.at[eidx.at[c + NBUF, 0]],
                                 rows.at[b], gsem.at[b])
        return carry
    lax.fori_loop(0, n_groups, _group, 0)

    for b in range(NBUF):
        c = (n_groups - 1) * NBUF + b
        pltpu.make_async_copy(srows.at[b], acc.at[eidx.at[c, 1]], ssem.at[b]).wait()

    plsc.subcore_barrier()

    @pl.when(cid == 0)
    def _():
        pltpu.sync_copy(acc.at[pl.ds(row0, ROWS_PER_TILE)],
                        out0_ref.at[pl.ds(row0, ROWS_PER_TILE)])

    @pl.when(cid == 1)
    def _():
        pltpu.sync_copy(acc.at[pl.ds(row0, ROWS_PER_TILE)],
                        out1_ref.at[pl.ds(row0, ROWS_PER_TILE)])


def _spmm(table, e3, w_flat):
    n_tiles = e3.shape[0]
    nc_max = -(-n_tiles // NW)
    nc_max = -(-nc_max // NBUF) * NBUF
    tail = n_tiles - (NW - 1) * nc_max
    assert 0 < tail <= nc_max and tail % NBUF == 0
    mesh = plsc.VectorSubcoreMesh(core_axis_name="c", subcore_axis_name="s")
    f = pl.kernel(
        functools.partial(_spmm_body, nc_max, tail),
        out_type=[jax.ShapeDtypeStruct((N_NODES, C1), jnp.float32),
                  jax.ShapeDtypeStruct((N_NODES, C1), jnp.float32)],
        mesh=mesh,
        scratch_types=[
            pltpu.VMEM_SHARED((N_NODES, C1), jnp.float32),
            pltpu.VMEM_SHARED((N_NODES, C1), jnp.float32),
            pltpu.VMEM((nc_max, 2, B), jnp.int32),
            pltpu.VMEM((nc_max * B,), jnp.float32),
            pltpu.VMEM((NBUF, B, C1), jnp.float32),
            pltpu.VMEM((NBUF, B, C1), jnp.float32),
            pltpu.VMEM((ROWS_PER_TILE, C1), jnp.float32),
            pltpu.SemaphoreType.DMA((NBUF,)),
            pltpu.SemaphoreType.DMA((NBUF,)),
        ],
        compiler_params=pltpu.CompilerParams(use_tc_tiling_on_sc=False),
    )
    return f(table, e3, w_flat)


def kernel(x, edge_index, edge_weight, W1, W2):
    e_total = edge_weight.shape[0]
    assert e_total % B == 0
    ei = edge_index.astype(jnp.int32)
    w = edge_weight.astype(jnp.float32)
    e3 = ei.reshape(2, e_total // B, B).transpose(1, 0, 2)

    p0l, p1l = _tc_xw1(x, W1)
    s1a, s1b = _spmm(p1l.reshape(N_NODES, C1), e3, w)
    hl, q0f = _tc_finish1(p0l.reshape(-1), s1a.reshape(-1), s1b.reshape(-1), W2)
    s2a, s2b = _spmm(hl.reshape(N_NODES, C1), e3, w)
    outf = _tc_finish2(q0f, s2a.reshape(-1), s2b.reshape(-1), W2[1])
    return outf.reshape(N_NODES, C2)
